# baseline (speedup 1.0000x reference)
import jax
import jax.numpy as jnp
from jax import lax
from jax.experimental import pallas as pl
from jax.experimental.pallas import tpu as pltpu
from jax.experimental.pallas import tpu_sc as plsc

N_NODES = 50000
DIM = 64
HALF = 32
N_LAYERS = 3
N_EDGES = 800000

N_TILES = 16
N_CORES = 2

SCAT = 128
EDGES_PER_TILE = 51200
CPT = EDGES_PER_TILE // SCAT
NE_PAD = EDGES_PER_TILE * N_TILES
NIDX = NE_PAD // SCAT

NRING = 4
NIRING = 8

ROWS_PER_TILE = 3200
N_PAD = ROWS_PER_TILE * N_TILES
RCH = 128
NRCH = ROWS_PER_TILE // RCH

ACC_OFF = 0
OUT_OFF = RCH
ZERO_OFF = 2 * RCH

DUMMY_ROW = N_NODES


def _body(col_hbm, row_hbm, emb_hbm, out_hbm, xbuf_hbm,
          acc, degacc, colb, rowb, ring, ones_v, d2_buf,
          gsem, ssem, isem):
    c = lax.axis_index("c")
    s = lax.axis_index("s")
    r0 = s * ROWS_PER_TILE
    g0 = c * N_PAD + r0

    def _rslot(m):
        return pl.ds(lax.rem(m, NRING) * SCAT, SCAT)

    def _zero_region(off):
        def _z(r, carry):
            ring[off + r, pl.ds(0, 16)] = jnp.zeros((16,), jnp.float32)
            ring[off + r, pl.ds(16, 16)] = jnp.zeros((16,), jnp.float32)
            return carry
        lax.fori_loop(0, RCH, _z, 0)

    for i in range(SCAT // 16):
        ones_v[pl.ds(i * 16, 16)] = jnp.full((16,), 1.0, jnp.float32)

    def _zd(i, carry):
        d2_buf[pl.ds(i * 16, 16)] = jnp.zeros((16,), jnp.float32)
        return carry
    lax.fori_loop(0, ROWS_PER_TILE // 16, _zd, 0)

    _zero_region(ZERO_OFF)

    def _init_chunk(k, carry):
        stage = ring.at[pl.ds(ACC_OFF, RCH)]
        pltpu.sync_copy(emb_hbm.at[pl.ds(g0 + k * RCH, RCH)], stage)
        pltpu.sync_copy(stage, xbuf_hbm.at[pl.ds(g0 + k * RCH, RCH)])
        pltpu.sync_copy(stage, out_hbm.at[pl.ds(g0 + k * RCH, RCH)])
        pltpu.sync_copy(ring.at[pl.ds(ZERO_OFF, RCH)],
                        acc.at[pl.ds(r0 + k * RCH, RCH)])
        return carry
    lax.fori_loop(0, NRCH, _init_chunk, 0)
    pltpu.sync_copy(d2_buf, degacc.at[pl.ds(r0, ROWS_PER_TILE)])
    plsc.subcore_barrier()

    for l in range(N_LAYERS):
        last = l == N_LAYERS - 1
        layer0 = l == 0
        cb0 = c * NIDX + s * CPT
        rb0 = s * CPT

        def _drain_rows(n_chunks, semm):
            pltpu.make_async_copy(emb_hbm.at[pl.ds(0, n_chunks * SCAT)],
                                  ring.at[pl.ds(0, n_chunks * SCAT)],
                                  semm).wait()

        def _drain_idx(n_rows, semm):
            pltpu.make_async_copy(row_hbm.at[pl.ds(0, n_rows)],
                                  rowb.at[pl.ds(0, n_rows)], semm).wait()

        pltpu.sync_copy(col_hbm.at[pl.ds(cb0, 2)], colb.at[pl.ds(0, 2)])
        pltpu.sync_copy(row_hbm.at[pl.ds(rb0, 2)], rowb.at[pl.ds(0, 2)])
        pltpu.async_copy(col_hbm.at[pl.ds(cb0 + 2, 2)],
                         colb.at[pl.ds(2, 2)], isem)
        pltpu.async_copy(row_hbm.at[pl.ds(rb0 + 2, 2)],
                         rowb.at[pl.ds(2, 2)], isem)
        for m in range(2):
            pltpu.async_copy(xbuf_hbm.at[colb.at[m]], ring.at[_rslot(m)],
                             gsem)

        NGRP = CPT // 2

        def _edge(g, carry, _layer0=layer0, _cb0=cb0, _rb0=rb0):
            a = 2 * g
            @pl.when(g > 0)
            def _():
                _drain_rows(2, ssem)
                if _layer0:
                    _drain_idx(2, ssem)
            @pl.when(g + 1 < NGRP)
            def _():
                _drain_idx(4, isem)
                for m in range(2):
                    bm = lax.rem(a + 2 + m, NIRING)
                    pltpu.async_copy(xbuf_hbm.at[colb.at[bm]],
                                     ring.at[_rslot(a + 2 + m)], gsem)
            _drain_rows(2, gsem)
            @pl.when(g + 2 < NGRP)
            def _():
                b4 = lax.rem(a + 4, NIRING)
                pltpu.async_copy(col_hbm.at[pl.ds(_cb0 + a + 4, 2)],
                                 colb.at[pl.ds(b4, 2)], isem)
                pltpu.async_copy(row_hbm.at[pl.ds(_rb0 + a + 4, 2)],
                                 rowb.at[pl.ds(b4, 2)], isem)
            for m in range(2):
                bm = lax.rem(a + m, NIRING)
                pltpu.async_copy(ring.at[_rslot(a + m)],
                                 acc.at[rowb.at[bm]], ssem, add=True)
                if _layer0:
                    pltpu.async_copy(ones_v, degacc.at[rowb.at[bm]],
                                     ssem, add=True)
            return carry
        lax.fori_loop(0, NGRP, _edge, 0)
        _drain_rows(2, ssem)
        if layer0:
            _drain_idx(2, ssem)
        plsc.subcore_barrier()

        if layer0:
            pltpu.sync_copy(degacc.at[pl.ds(r0, ROWS_PER_TILE)], d2_buf)

            def _d2(i, carry):
                d = d2_buf[pl.ds(i * 16, 16)]
                d2_buf[pl.ds(i * 16, 16)] = jnp.where(
                    d > 0.0, 1.0 / d, jnp.zeros((16,), jnp.float32))
                return carry
            lax.fori_loop(0, ROWS_PER_TILE // 16, _d2, 0)

        if not last:
            _zero_region(ZERO_OFF)

        def _scale_chunk(k, carry, _last=last):
            gr = g0 + k * RCH
            ar = r0 + k * RCH
            pltpu.sync_copy(acc.at[pl.ds(ar, RCH)],
                            ring.at[pl.ds(ACC_OFF, RCH)])
            pltpu.sync_copy(out_hbm.at[pl.ds(gr, RCH)],
                            ring.at[pl.ds(OUT_OFF, RCH)])

            def _srow(i, carry2, _k=k):
                base = i * 16
                dvec = d2_buf[pl.ds(_k * RCH + base, 16)]
                for rr in range(16):
                    r = base + rr
                    dd = dvec[rr]
                    for h in range(HALF // 16):
                        v = ring[ACC_OFF + r, pl.ds(h * 16, 16)] * dd
                        ring[ACC_OFF + r, pl.ds(h * 16, 16)] = v
                        o = ring[OUT_OFF + r, pl.ds(h * 16, 16)] + v
                        if _last:
                            o = o * 0.25
                        ring[OUT_OFF + r, pl.ds(h * 16, 16)] = o
                return carry2
            lax.fori_loop(0, RCH // 16, _srow, 0)
            pltpu.sync_copy(ring.at[pl.ds(OUT_OFF, RCH)],
                            out_hbm.at[pl.ds(gr, RCH)])
            if not _last:
                pltpu.sync_copy(ring.at[pl.ds(ACC_OFF, RCH)],
                                xbuf_hbm.at[pl.ds(gr, RCH)])
                pltpu.sync_copy(ring.at[pl.ds(ZERO_OFF, RCH)],
                                acc.at[pl.ds(ar, RCH)])
            return carry
        lax.fori_loop(0, NRCH, _scale_chunk, 0)
        if not last:
            plsc.subcore_barrier()


@jax.jit
def kernel(edge_index, embedding_weight):
    row = edge_index[0].astype(jnp.int32)
    col = edge_index[1].astype(jnp.int32)
    npad = NE_PAD - N_EDGES
    col, row = lax.sort_key_val(col, row)
    row_p = jnp.concatenate(
        [row, jnp.full((npad,), DUMMY_ROW, jnp.int32)]).reshape(-1, SCAT)
    col_p = jnp.concatenate([col, jnp.zeros((npad,), jnp.int32)])
    col2 = jnp.concatenate([col_p, col_p + N_PAD]).reshape(-1, SCAT)

    zrows = jnp.zeros((N_PAD - N_NODES, HALF), jnp.float32)
    emb = jnp.concatenate([
        embedding_weight[:, :HALF], zrows,
        embedding_weight[:, HALF:], zrows], axis=0)

    mesh = plsc.VectorSubcoreMesh(core_axis_name="c", subcore_axis_name="s")
    out, _ = pl.kernel(
        _body,
        mesh=mesh,
        compiler_params=pltpu.CompilerParams(use_tc_tiling_on_sc=False),
        out_type=(
            jax.ShapeDtypeStruct((2 * N_PAD, HALF), jnp.float32),
            jax.ShapeDtypeStruct((2 * N_PAD, HALF), jnp.float32),
        ),
        scratch_types=[
            pltpu.VMEM_SHARED((N_PAD, HALF), jnp.float32),
            pltpu.VMEM_SHARED((N_PAD,), jnp.float32),
            pltpu.VMEM((NIRING, SCAT), jnp.int32),
            pltpu.VMEM((NIRING, SCAT), jnp.int32),
            pltpu.VMEM((NRING * SCAT, HALF), jnp.float32),
            pltpu.VMEM((SCAT,), jnp.float32),
            pltpu.VMEM((ROWS_PER_TILE,), jnp.float32),
            pltpu.SemaphoreType.DMA,
            pltpu.SemaphoreType.DMA,
            pltpu.SemaphoreType.DMA,
        ],
    )(col2, row_p, emb)
    return jnp.concatenate(
        [out[:N_NODES], out[N_PAD:N_PAD + N_NODES]], axis=1)


# speedup vs baseline: 1.8961x; 1.8961x over previous
import jax
import jax.numpy as jnp
from jax import lax
from jax.experimental import pallas as pl
from jax.experimental.pallas import tpu as pltpu
from jax.experimental.pallas import tpu_sc as plsc

N_NODES = 50000
DIM = 64
HALF = 32
N_LAYERS = 3
N_EDGES = 800000

N_TILES = 16
N_CORES = 2

SCAT = 128
EDGES_PER_TILE = 51200
CPT = EDGES_PER_TILE // SCAT
NE_PAD = EDGES_PER_TILE * N_TILES
NIDX = NE_PAD // SCAT

NRING = 4
NIRING = 8

ROWS_PER_TILE = 3200
N_PAD = ROWS_PER_TILE * N_TILES
RCH = 128
NRCH = ROWS_PER_TILE // RCH

ACC_OFF = 0
OUT_OFF = RCH
ZERO_OFF = 2 * RCH

DUMMY_ROW = N_NODES


def _body(col_hbm, row_hbm, emb_hbm, out_hbm, xbuf_hbm,
          acc, degacc, colb, rowb, ring, ones_v, d2_buf,
          gsem, ssem, isem):
    c = lax.axis_index("c")
    s = lax.axis_index("s")
    r0 = s * ROWS_PER_TILE
    g0 = c * N_PAD + r0

    def _rslot(m):
        return pl.ds(lax.rem(m, NRING) * SCAT, SCAT)

    def _zero_region(off):
        def _z(r, carry):
            ring[off + r, pl.ds(0, 16)] = jnp.zeros((16,), jnp.float32)
            ring[off + r, pl.ds(16, 16)] = jnp.zeros((16,), jnp.float32)
            return carry
        lax.fori_loop(0, RCH, _z, 0)

    for i in range(SCAT // 16):
        ones_v[pl.ds(i * 16, 16)] = jnp.full((16,), 1.0, jnp.float32)

    def _zd(i, carry):
        d2_buf[pl.ds(i * 16, 16)] = jnp.zeros((16,), jnp.float32)
        return carry
    lax.fori_loop(0, ROWS_PER_TILE // 16, _zd, 0)

    _zero_region(ZERO_OFF)

    def _init_chunk(k, carry):
        stage = ring.at[pl.ds(ACC_OFF, RCH)]
        pltpu.sync_copy(emb_hbm.at[pl.ds(g0 + k * RCH, RCH)], stage)
        pltpu.sync_copy(stage, xbuf_hbm.at[pl.ds(g0 + k * RCH, RCH)])
        pltpu.sync_copy(stage, out_hbm.at[pl.ds(g0 + k * RCH, RCH)])
        pltpu.sync_copy(ring.at[pl.ds(ZERO_OFF, RCH)],
                        acc.at[pl.ds(r0 + k * RCH, RCH)])
        return carry
    lax.fori_loop(0, NRCH, _init_chunk, 0)
    pltpu.sync_copy(d2_buf, degacc.at[pl.ds(r0, ROWS_PER_TILE)])
    plsc.subcore_barrier()

    for l in range(N_LAYERS):
        last = l == N_LAYERS - 1
        layer0 = l == 0
        cb0 = c * NE_PAD + s * EDGES_PER_TILE
        rb0 = s * CPT

        def _drain_rows(n_chunks, semm):
            pltpu.make_async_copy(emb_hbm.at[pl.ds(0, n_chunks * SCAT)],
                                  ring.at[pl.ds(0, n_chunks * SCAT)],
                                  semm).wait()

        def _drain_idx(n_rows, semm):
            pltpu.make_async_copy(row_hbm.at[pl.ds(0, n_rows)],
                                  rowb.at[pl.ds(0, n_rows)], semm).wait()

        GSZ = 2 * SCAT
        pltpu.sync_copy(col_hbm.at[pl.ds(cb0, GSZ)], colb.at[pl.ds(0, GSZ)])
        pltpu.sync_copy(row_hbm.at[pl.ds(rb0, 2)], rowb.at[pl.ds(0, 2)])
        pltpu.async_copy(col_hbm.at[pl.ds(cb0 + GSZ, GSZ)],
                         colb.at[pl.ds(GSZ, GSZ)], isem)
        pltpu.async_copy(row_hbm.at[pl.ds(rb0 + 2, 2)],
                         rowb.at[pl.ds(2, 2)], isem)
        pltpu.async_copy(xbuf_hbm.at[colb.at[pl.ds(0, GSZ)]],
                         ring.at[pl.ds(0, GSZ)], gsem)

        NGRP = CPT // 2

        def _edge(g, carry, _layer0=layer0, _cb0=cb0, _rb0=rb0):
            a = 2 * g
            ghalf = lax.rem(g, 2) * GSZ
            @pl.when(g > 0)
            def _():
                _drain_rows(2, ssem)
                if _layer0:
                    _drain_idx(2, ssem)
            @pl.when(g + 1 < NGRP)
            def _():
                _drain_idx(4, isem)
                bg1 = lax.rem(g + 1, 4) * GSZ
                pltpu.async_copy(xbuf_hbm.at[colb.at[pl.ds(bg1, GSZ)]],
                                 ring.at[pl.ds(lax.rem(g + 1, 2) * GSZ,
                                               GSZ)], gsem)
            _drain_rows(2, gsem)
            @pl.when(g + 2 < NGRP)
            def _():
                bc2 = lax.rem(g + 2, 4) * GSZ
                pltpu.async_copy(col_hbm.at[pl.ds(_cb0 + (a + 4) * SCAT,
                                                  GSZ)],
                                 colb.at[pl.ds(bc2, GSZ)], isem)
                br2 = lax.rem(a + 4, NIRING)
                pltpu.async_copy(row_hbm.at[pl.ds(_rb0 + a + 4, 2)],
                                 rowb.at[pl.ds(br2, 2)], isem)
            for m in range(2):
                bm = lax.rem(a + m, NIRING)
                pltpu.async_copy(ring.at[pl.ds(ghalf + m * SCAT, SCAT)],
                                 acc.at[rowb.at[bm]], ssem, add=True)
                if _layer0:
                    pltpu.async_copy(ones_v, degacc.at[rowb.at[bm]],
                                     ssem, add=True)
            return carry
        lax.fori_loop(0, NGRP, _edge, 0)
        _drain_rows(2, ssem)
        if layer0:
            _drain_idx(2, ssem)
        plsc.subcore_barrier()

        if layer0:
            pltpu.sync_copy(degacc.at[pl.ds(r0, ROWS_PER_TILE)], d2_buf)

            def _d2(i, carry):
                d = d2_buf[pl.ds(i * 16, 16)]
                d2_buf[pl.ds(i * 16, 16)] = jnp.where(
                    d > 0.0, 1.0 / d, jnp.zeros((16,), jnp.float32))
                return carry
            lax.fori_loop(0, ROWS_PER_TILE // 16, _d2, 0)

        if not last:
            _zero_region(ZERO_OFF)

        def _scale_chunk(k, carry, _last=last):
            gr = g0 + k * RCH
            ar = r0 + k * RCH
            pltpu.sync_copy(acc.at[pl.ds(ar, RCH)],
                            ring.at[pl.ds(ACC_OFF, RCH)])
            pltpu.sync_copy(out_hbm.at[pl.ds(gr, RCH)],
                            ring.at[pl.ds(OUT_OFF, RCH)])

            def _srow(i, carry2, _k=k):
                base = i * 16
                dvec = d2_buf[pl.ds(_k * RCH + base, 16)]
                for rr in range(16):
                    r = base + rr
                    dd = dvec[rr]
                    for h in range(HALF // 16):
                        v = ring[ACC_OFF + r, pl.ds(h * 16, 16)] * dd
                        ring[ACC_OFF + r, pl.ds(h * 16, 16)] = v
                        o = ring[OUT_OFF + r, pl.ds(h * 16, 16)] + v
                        if _last:
                            o = o * 0.25
                        ring[OUT_OFF + r, pl.ds(h * 16, 16)] = o
                return carry2
            lax.fori_loop(0, RCH // 16, _srow, 0)
            pltpu.sync_copy(ring.at[pl.ds(OUT_OFF, RCH)],
                            out_hbm.at[pl.ds(gr, RCH)])
            if not _last:
                pltpu.sync_copy(ring.at[pl.ds(ACC_OFF, RCH)],
                                xbuf_hbm.at[pl.ds(gr, RCH)])
                pltpu.sync_copy(ring.at[pl.ds(ZERO_OFF, RCH)],
                                acc.at[pl.ds(ar, RCH)])
            return carry
        lax.fori_loop(0, NRCH, _scale_chunk, 0)
        if not last:
            plsc.subcore_barrier()


@jax.jit
def kernel(edge_index, embedding_weight):
    row = edge_index[0].astype(jnp.int32)
    col = edge_index[1].astype(jnp.int32)
    npad = NE_PAD - N_EDGES
    row_p = jnp.concatenate(
        [row, jnp.full((npad,), DUMMY_ROW, jnp.int32)]).reshape(-1, SCAT)
    col_p = jnp.concatenate([col, jnp.zeros((npad,), jnp.int32)])
    col2 = jnp.concatenate([col_p, col_p + N_PAD])

    zrows = jnp.zeros((N_PAD - N_NODES, HALF), jnp.float32)
    emb = jnp.concatenate([
        embedding_weight[:, :HALF], zrows,
        embedding_weight[:, HALF:], zrows], axis=0)

    mesh = plsc.VectorSubcoreMesh(core_axis_name="c", subcore_axis_name="s")
    out, _ = pl.kernel(
        _body,
        mesh=mesh,
        compiler_params=pltpu.CompilerParams(use_tc_tiling_on_sc=False),
        out_type=(
            jax.ShapeDtypeStruct((2 * N_PAD, HALF), jnp.float32),
            jax.ShapeDtypeStruct((2 * N_PAD, HALF), jnp.float32),
        ),
        scratch_types=[
            pltpu.VMEM_SHARED((N_PAD, HALF), jnp.float32),
            pltpu.VMEM_SHARED((N_PAD,), jnp.float32),
            pltpu.VMEM((4 * 2 * SCAT,), jnp.int32),
            pltpu.VMEM((NIRING, SCAT), jnp.int32),
            pltpu.VMEM((NRING * SCAT, HALF), jnp.float32),
            pltpu.VMEM((SCAT,), jnp.float32),
            pltpu.VMEM((ROWS_PER_TILE,), jnp.float32),
            pltpu.SemaphoreType.DMA,
            pltpu.SemaphoreType.DMA,
            pltpu.SemaphoreType.DMA,
        ],
    )(col2, row_p, emb)
    return jnp.concatenate(
        [out[:N_NODES], out[N_PAD:N_PAD + N_NODES]], axis=1)
